# initial kernel scaffold (unmeasured)
import jax
import jax.numpy as jnp
from jax import lax
from jax.experimental import pallas as pl
from jax.experimental.pallas import tpu as pltpu

N_DEV = 16
B, SQ, SKV = 2, 512, 512
HQ_PER, DH = 8, 64
D_MODEL = 768
ROWS = B * SQ
CHUNK = ROWS // N_DEV
N_STEPS = 2 * (N_DEV - 1)


def kernel(x, Wq, K_ext, V_ext, Wo):
    my = lax.axis_index("i")
    K_sh = lax.dynamic_slice_in_dim(K_ext, my * HQ_PER, HQ_PER, axis=2)
    V_sh = lax.dynamic_slice_in_dim(V_ext, my * HQ_PER, HQ_PER, axis=2)

    def body(x_ref, wq_ref, k_ref, v_ref, wo_ref, out_ref,
             ctx_ref, comm_ref, send_sems, recv_sems, credit_sem):
        my_pos = lax.axis_index("i")
        left = lax.rem(my_pos + N_DEV - 1, N_DEV)
        right = lax.rem(my_pos + 1, N_DEV)

        barrier_sem = pltpu.get_barrier_semaphore()
        for nbr in (left, right):
            pl.semaphore_signal(barrier_sem, inc=1, device_id=(nbr,),
                                device_id_type=pl.DeviceIdType.MESH)
        pl.semaphore_wait(barrier_sem, 2)

        qi = lax.broadcasted_iota(jnp.int32, (SQ, SKV), 0)
        ki = lax.broadcasted_iota(jnp.int32, (SQ, SKV), 1)
        mask = (jnp.abs(qi - ki) <= 128) | (ki < 32) | (qi < 32)
        neg = jnp.float32(-1e9)

        for b in range(B):
            q_all = jnp.dot(x_ref[b, :, :], wq_ref[:, :],
                            preferred_element_type=jnp.float32)
            for h in range(HQ_PER):
                q_h = q_all[:, h * DH:(h + 1) * DH]
                k_h = k_ref[b, :, h, :]
                v_h = v_ref[b, :, h, :]
                s = lax.dot_general(q_h, k_h, (((1,), (1,)), ((), ())),
                                    preferred_element_type=jnp.float32)
                s = s * jnp.float32(0.125)
                s = jnp.where(mask, s, neg)
                m = jnp.max(s, axis=1, keepdims=True)
                w = jnp.exp(s - m)
                w = w / jnp.sum(w, axis=1, keepdims=True)
                ctx_ref[:, h * DH:(h + 1) * DH] = jnp.dot(
                    w, v_h, preferred_element_type=jnp.float32)
            out_ref[b * SQ:(b + 1) * SQ, :] = jnp.dot(
                ctx_ref[:, :], wo_ref[:, :],
                preferred_element_type=jnp.float32)

        for s_i in range(N_STEPS):
            slot = s_i % 2
            if s_i >= 2:
                pl.semaphore_wait(credit_sem, 1)
            if s_i < N_DEV - 1:
                sc = lax.rem(my_pos - s_i + 2 * N_DEV, N_DEV)
                rc = lax.rem(my_pos - s_i - 1 + 2 * N_DEV, N_DEV)
            else:
                ag = s_i - (N_DEV - 1)
                sc = lax.rem(my_pos + 1 - ag + 2 * N_DEV, N_DEV)
                rc = lax.rem(my_pos - ag + 2 * N_DEV, N_DEV)
            rdma = pltpu.make_async_remote_copy(
                src_ref=out_ref.at[pl.ds(sc * CHUNK, CHUNK), :],
                dst_ref=comm_ref.at[slot],
                send_sem=send_sems.at[slot],
                recv_sem=recv_sems.at[slot],
                device_id=(right,),
                device_id_type=pl.DeviceIdType.MESH,
            )
            rdma.start()
            rdma.wait()
            inc = comm_ref[slot]
            if s_i < N_DEV - 1:
                acc = pl.load(out_ref, (pl.ds(rc * CHUNK, CHUNK), slice(None)))
                pl.store(out_ref, (pl.ds(rc * CHUNK, CHUNK), slice(None)),
                         acc + inc)
            else:
                pl.store(out_ref, (pl.ds(rc * CHUNK, CHUNK), slice(None)), inc)
            if s_i < N_STEPS - 2:
                pl.semaphore_signal(credit_sem, inc=1, device_id=(left,),
                                    device_id_type=pl.DeviceIdType.MESH)

    out = pl.pallas_call(
        body,
        out_shape=jax.ShapeDtypeStruct((ROWS, D_MODEL), jnp.float32),
        in_specs=[pl.BlockSpec(memory_space=pltpu.VMEM)] * 5,
        out_specs=pl.BlockSpec(memory_space=pltpu.VMEM),
        scratch_shapes=[
            pltpu.VMEM((SQ, HQ_PER * DH), jnp.float32),
            pltpu.VMEM((2, CHUNK, D_MODEL), jnp.float32),
            pltpu.SemaphoreType.DMA((2,)),
            pltpu.SemaphoreType.DMA((2,)),
            pltpu.SemaphoreType.REGULAR,
        ],
        compiler_params=pltpu.CompilerParams(collective_id=0),
    )(x, Wq, K_sh, V_sh, Wo)
    return out.reshape(B, SQ, D_MODEL)


# baseline (device time: 222147 ns/iter reference)
import jax
import jax.numpy as jnp
from jax import lax
from jax.experimental import pallas as pl
from jax.experimental.pallas import tpu as pltpu

N_DEV = 16
B, SQ, SKV = 2, 512, 512
HQ_PER, DH = 8, 64
D_MODEL = 768
ROWS = B * SQ
CHUNK = ROWS // N_DEV
N_STEPS = 2 * (N_DEV - 1)


def kernel(x, Wq, K_ext, V_ext, Wo):
    my = lax.axis_index("i")
    K_sh = lax.dynamic_slice_in_dim(K_ext, my * HQ_PER, HQ_PER, axis=2)
    V_sh = lax.dynamic_slice_in_dim(V_ext, my * HQ_PER, HQ_PER, axis=2)

    def body(x_ref, wq_ref, k_ref, v_ref, wo_ref, out_ref,
             ctx_ref, comm_ref, send_sems, recv_sems, credit_sem):
        my_pos = lax.axis_index("i")
        left = lax.rem(my_pos + N_DEV - 1, N_DEV)
        right = lax.rem(my_pos + 1, N_DEV)

        barrier_sem = pltpu.get_barrier_semaphore()
        for nbr in (left, right):
            pl.semaphore_signal(barrier_sem, inc=1, device_id=(nbr,),
                                device_id_type=pl.DeviceIdType.MESH)
        pl.semaphore_wait(barrier_sem, 2)

        qi = lax.broadcasted_iota(jnp.int32, (SQ, SKV), 0)
        ki = lax.broadcasted_iota(jnp.int32, (SQ, SKV), 1)
        mask = (jnp.abs(qi - ki) <= 128) | (ki < 32) | (qi < 32)
        neg = jnp.float32(-1e9)

        for b in range(B):
            q_all = jnp.dot(x_ref[b, :, :], wq_ref[:, :],
                            preferred_element_type=jnp.float32)
            for h in range(HQ_PER):
                q_h = q_all[:, h * DH:(h + 1) * DH]
                k_h = k_ref[b, :, h, :]
                v_h = v_ref[b, :, h, :]
                s = lax.dot_general(q_h, k_h, (((1,), (1,)), ((), ())),
                                    preferred_element_type=jnp.float32)
                s = s * jnp.float32(0.125)
                s = jnp.where(mask, s, neg)
                m = jnp.max(s, axis=1, keepdims=True)
                w = jnp.exp(s - m)
                w = w / jnp.sum(w, axis=1, keepdims=True)
                ctx_ref[:, h * DH:(h + 1) * DH] = jnp.dot(
                    w, v_h, preferred_element_type=jnp.float32)
            out_ref[b * SQ:(b + 1) * SQ, :] = jnp.dot(
                ctx_ref[:, :], wo_ref[:, :],
                preferred_element_type=jnp.float32)

        for s_i in range(N_STEPS):
            slot = s_i % 2
            if s_i >= 2:
                pl.semaphore_wait(credit_sem, 1)
            if s_i < N_DEV - 1:
                sc = lax.rem(my_pos - s_i + 2 * N_DEV, N_DEV)
                rc = lax.rem(my_pos - s_i - 1 + 2 * N_DEV, N_DEV)
            else:
                ag = s_i - (N_DEV - 1)
                sc = lax.rem(my_pos + 1 - ag + 2 * N_DEV, N_DEV)
                rc = lax.rem(my_pos - ag + 2 * N_DEV, N_DEV)
            rdma = pltpu.make_async_remote_copy(
                src_ref=out_ref.at[pl.ds(sc * CHUNK, CHUNK), :],
                dst_ref=comm_ref.at[slot],
                send_sem=send_sems.at[slot],
                recv_sem=recv_sems.at[slot],
                device_id=(right,),
                device_id_type=pl.DeviceIdType.MESH,
            )
            rdma.start()
            rdma.wait()
            inc = comm_ref[slot]
            if s_i < N_DEV - 1:
                acc = out_ref[pl.ds(rc * CHUNK, CHUNK), :]
                out_ref[pl.ds(rc * CHUNK, CHUNK), :] = acc + inc
            else:
                out_ref[pl.ds(rc * CHUNK, CHUNK), :] = inc
            if s_i < N_STEPS - 2:
                pl.semaphore_signal(credit_sem, inc=1, device_id=(left,),
                                    device_id_type=pl.DeviceIdType.MESH)

    out = pl.pallas_call(
        body,
        out_shape=jax.ShapeDtypeStruct((ROWS, D_MODEL), jnp.float32),
        in_specs=[pl.BlockSpec(memory_space=pltpu.VMEM)] * 5,
        out_specs=pl.BlockSpec(memory_space=pltpu.VMEM),
        scratch_shapes=[
            pltpu.VMEM((SQ, HQ_PER * DH), jnp.float32),
            pltpu.VMEM((2, CHUNK, D_MODEL), jnp.float32),
            pltpu.SemaphoreType.DMA((2,)),
            pltpu.SemaphoreType.DMA((2,)),
            pltpu.SemaphoreType.REGULAR,
        ],
        compiler_params=pltpu.CompilerParams(collective_id=0),
    )(x, Wq, K_sh, V_sh, Wo)
    return out.reshape(B, SQ, D_MODEL)
